# baseline (speedup 1.0000x reference)
import jax
import jax.numpy as jnp
from jax.experimental import pallas as pl
from jax.experimental.pallas import tpu as pltpu

F32 = jnp.float32


def _remb_kernel(x_ref, wr_ref, br_ref, o_ref):
    T, _ = x_ref.shape
    NC = o_ref.shape[0]
    CS = T // NC
    cm = jnp.mean(x_ref[:].reshape(NC, CS, -1), axis=1)
    re = jnp.dot(cm, wr_ref[:], preferred_element_type=F32) + br_ref[:]
    nrm = jnp.sqrt(jnp.sum(re * re, axis=-1, keepdims=True))
    o_ref[:] = re / jnp.maximum(nrm, 1e-12)


def _sel_kernel(x_ref, wg_ref, bg_ref, ren_ref, o_ref, *, topb):
    qr = jnp.dot(x_ref[:], wg_ref[:], preferred_element_type=F32) + bg_ref[:]
    nrm = jnp.sqrt(jnp.sum(qr * qr, axis=-1, keepdims=True))
    qn = qr / jnp.maximum(nrm, 1e-12)
    s = jax.lax.dot_general(qn, ren_ref[:], (((1,), (1,)), ((), ())),
                            preferred_element_type=F32)
    NC = s.shape[1]
    sm = s[:, None, :]
    sn = s[:, :, None]
    gt = (sm > sn).astype(F32)
    n_idx = jax.lax.broadcasted_iota(jnp.int32, (NC, NC), 0)
    m_idx = jax.lax.broadcasted_iota(jnp.int32, (NC, NC), 1)
    mlt = (m_idx < n_idx)[None, :, :]
    eq = (sm == sn) & mlt
    rank = jnp.sum(gt, axis=2) + jnp.sum(eq.astype(F32), axis=2)
    o_ref[:] = (rank < topb).astype(F32)


def _qkv_kernel(x_ref, w_ref, b_ref, o_ref):
    o_ref[0] = jnp.dot(x_ref[:], w_ref[0], preferred_element_type=F32) + b_ref[0]


def _attn_kernel(q_ref, k_ref, v_ref, sel_ref, o_ref, mask_ref, *,
                 bq, lw, cs, scale):
    i = pl.program_id(0)
    h = pl.program_id(1)
    T = k_ref.shape[1]
    NC = sel_ref.shape[1]

    @pl.when(h == 0)
    def _build_mask():
        ci = jax.lax.broadcasted_iota(jnp.int32, (NC, T), 0)
        si = jax.lax.broadcasted_iota(jnp.int32, (NC, T), 1) // cs
        E = (ci == si).astype(F32)
        selx = jnp.dot(sel_ref[:], E, preferred_element_type=F32)
        t_ids = i * bq + jax.lax.broadcasted_iota(jnp.int32, (bq, T), 0)
        s_ids = jax.lax.broadcasted_iota(jnp.int32, (bq, T), 1)
        allowed = (t_ids >= s_ids) & (((t_ids - s_ids) < lw) | (selx > 0.5))
        mask_ref[:] = jnp.where(allowed, 0.0, -1e9).astype(F32)

    s = jax.lax.dot_general(q_ref[0], k_ref[0], (((1,), (1,)), ((), ())),
                            preferred_element_type=F32)
    s = s * scale + mask_ref[:]
    mx = jnp.max(s, axis=1, keepdims=True)
    p = jnp.exp(s - mx)
    l = jnp.sum(p, axis=1, keepdims=True)
    y = jnp.dot(p, v_ref[0], preferred_element_type=F32)
    o_ref[0] = y / l


def _proj_kernel(y_ref, w_ref, b_ref, o_ref):
    h = pl.program_id(1)

    @pl.when(h == 0)
    def _init():
        o_ref[:] = jnp.broadcast_to(b_ref[:], o_ref.shape)

    o_ref[:] += jnp.dot(y_ref[0], w_ref[0], preferred_element_type=F32)


def kernel(x, W_attn, b_attn, W_proj, b_proj, W_router, b_router, W_gate, b_gate):
    B, T, C = x.shape
    NH = 12
    HD = C // NH
    RD = W_router.shape[1]
    CS = 64
    NC = T // CS
    TOPB = 8
    LW = 256
    scale = 1.0 / (HD ** 0.5)

    x2 = x.reshape(T, C)
    b_router2 = b_router.reshape(1, RD)
    b_gate2 = b_gate.reshape(1, RD)
    Wa3 = W_attn.reshape(C, 3 * NH, HD).transpose(1, 0, 2)
    ba3 = b_attn.reshape(3 * NH, 1, HD)
    Wp3 = W_proj.reshape(NH, HD, C)
    b_proj2 = b_proj.reshape(1, C)

    ren = pl.pallas_call(
        _remb_kernel,
        out_shape=jax.ShapeDtypeStruct((NC, RD), F32),
    )(x2, W_router, b_router2)

    BTS = 512
    sel = pl.pallas_call(
        lambda *a: _sel_kernel(*a, topb=TOPB),
        grid=(T // BTS,),
        in_specs=[
            pl.BlockSpec((BTS, C), lambda i: (i, 0)),
            pl.BlockSpec((C, RD), lambda i: (0, 0)),
            pl.BlockSpec((1, RD), lambda i: (0, 0)),
            pl.BlockSpec((NC, RD), lambda i: (0, 0)),
        ],
        out_specs=pl.BlockSpec((BTS, NC), lambda i: (i, 0)),
        out_shape=jax.ShapeDtypeStruct((T, NC), F32),
    )(x2, W_gate, b_gate2, ren)

    BT = 256
    NB = T // BT
    qkv = pl.pallas_call(
        _qkv_kernel,
        grid=(NB, 3 * NH),
        in_specs=[
            pl.BlockSpec((BT, C), lambda i, j: (i, 0)),
            pl.BlockSpec((1, C, HD), lambda i, j: (j, 0, 0)),
            pl.BlockSpec((1, 1, HD), lambda i, j: (j, 0, 0)),
        ],
        out_specs=pl.BlockSpec((1, BT, HD), lambda i, j: (j, i, 0)),
        out_shape=jax.ShapeDtypeStruct((3 * NH, T, HD), F32),
    )(x2, Wa3, ba3)

    BQ = 256
    NBQ = T // BQ
    y = pl.pallas_call(
        lambda *a: _attn_kernel(*a, bq=BQ, lw=LW, cs=CS, scale=scale),
        grid=(NBQ, NH),
        in_specs=[
            pl.BlockSpec((1, BQ, HD), lambda i, h: (h, i, 0)),
            pl.BlockSpec((1, T, HD), lambda i, h: (NH + h, 0, 0)),
            pl.BlockSpec((1, T, HD), lambda i, h: (2 * NH + h, 0, 0)),
            pl.BlockSpec((BQ, NC), lambda i, h: (i, 0)),
        ],
        out_specs=pl.BlockSpec((1, BQ, HD), lambda i, h: (h, i, 0)),
        out_shape=jax.ShapeDtypeStruct((NH, T, HD), F32),
        scratch_shapes=[pltpu.VMEM((BQ, T), F32)],
    )(qkv, qkv, qkv, sel)

    out = pl.pallas_call(
        _proj_kernel,
        grid=(NB, NH),
        in_specs=[
            pl.BlockSpec((1, BT, HD), lambda i, h: (h, i, 0)),
            pl.BlockSpec((1, HD, C), lambda i, h: (h, 0, 0)),
            pl.BlockSpec((1, C), lambda i, h: (0, 0)),
        ],
        out_specs=pl.BlockSpec((BT, C), lambda i, h: (i, 0)),
        out_shape=jax.ShapeDtypeStruct((T, C), F32),
    )(y, Wp3, b_proj2)

    return out.reshape(B, T, C)
